# initial kernel scaffold (unmeasured)
import jax
import jax.numpy as jnp
from jax import lax
from jax.experimental import pallas as pl
from jax.experimental.pallas import tpu as pltpu

N_DEV = 8
SQ = 1024
SKV = 1024
HQ = 8
DH = 128
DM = 1024
SCALE = 0.08838834764831843


def kernel(x, Wq, K_ext, V_ext, Wo):
    idx = lax.axis_index("i")
    x2 = x[0]
    K = lax.dynamic_index_in_dim(K_ext, idx, axis=0, keepdims=False)
    V = lax.dynamic_index_in_dim(V_ext, idx, axis=0, keepdims=False)
    K = K.transpose(1, 0, 2)
    V = V.transpose(1, 0, 2)

    def body(x_ref, wq_ref, k_ref, v_ref, wo_ref, out_ref,
             comm_ref, k_buf, v_buf, send_sems, recv_sems, local_sems,
             credit_sem):
        my = lax.axis_index("i")
        left = lax.rem(my + (N_DEV - 1), N_DEV)
        right = lax.rem(my + 1, N_DEV)

        barrier_sem = pltpu.get_barrier_semaphore()
        for nbr in (left, right):
            pl.semaphore_signal(barrier_sem, inc=1, device_id=(nbr,),
                                device_id_type=pl.DeviceIdType.MESH)
        pl.semaphore_wait(barrier_sem, 2)

        qi = lax.broadcasted_iota(jnp.int32, (SQ, SKV), 0)
        ki = lax.broadcasted_iota(jnp.int32, (SQ, SKV), 1)
        mask = (jnp.abs(qi - ki) <= 128) | (ki < 32) | (qi < 32)
        bias = jnp.where(mask, 0.0, -1e9).astype(jnp.float32)

        comm_ref[0, 0] = wq_ref[...]
        comm_ref[0, 1] = wo_ref[...]

        for h in range(N_DEV):
            slot = h % 2
            hb = lax.rem(my + (N_DEV - h), N_DEV)

            if h > 0:
                in_rdma = pltpu.make_async_remote_copy(
                    src_ref=comm_ref.at[slot],
                    dst_ref=comm_ref.at[slot],
                    send_sem=send_sems.at[slot],
                    recv_sem=recv_sems.at[slot],
                    device_id=(left,),
                    device_id_type=pl.DeviceIdType.MESH,
                )
                in_rdma.wait_recv()

            if h < N_DEV - 1:
                if h >= 1:
                    pl.semaphore_wait(credit_sem, 1)
                out_rdma = pltpu.make_async_remote_copy(
                    src_ref=comm_ref.at[slot],
                    dst_ref=comm_ref.at[(h + 1) % 2],
                    send_sem=send_sems.at[slot],
                    recv_sem=recv_sems.at[(h + 1) % 2],
                    device_id=(right,),
                    device_id_type=pl.DeviceIdType.MESH,
                )
                out_rdma.start()

            kcp = pltpu.make_async_copy(
                k_ref.at[pl.ds(hb * HQ, HQ)], k_buf.at[slot], local_sems.at[0])
            vcp = pltpu.make_async_copy(
                v_ref.at[pl.ds(hb * HQ, HQ)], v_buf.at[slot], local_sems.at[1])
            kcp.start()
            vcp.start()
            kcp.wait()
            vcp.wait()

            q = jnp.dot(x2_v := x_ref[...], comm_ref[slot, 0],
                        preferred_element_type=jnp.float32)
            del x2_v
            ctx_parts = []
            for head in range(HQ):
                qh = q[:, head * DH:(head + 1) * DH]
                kh = k_buf[slot, head]
                s = lax.dot_general(
                    qh, kh, (((1,), (1,)), ((), ())),
                    preferred_element_type=jnp.float32) * SCALE + bias
                m = jnp.max(s, axis=1, keepdims=True)
                w = jnp.exp(s - m)
                w = w / jnp.sum(w, axis=1, keepdims=True)
                ctx_parts.append(
                    jnp.dot(w, v_buf[slot, head],
                            preferred_element_type=jnp.float32))
            ctx = jnp.concatenate(ctx_parts, axis=1)
            partial = jnp.dot(ctx, comm_ref[slot, 1],
                              preferred_element_type=jnp.float32)
            if h == 0:
                out_ref[...] = partial
            else:
                out_ref[...] = out_ref[...] + partial

            if h < N_DEV - 1:
                out_rdma.wait_send()
            if h <= N_DEV - 3:
                pl.semaphore_signal(credit_sem, inc=1, device_id=(left,),
                                    device_id_type=pl.DeviceIdType.MESH)

    out = pl.pallas_call(
        body,
        out_shape=jax.ShapeDtypeStruct((SQ, DM), jnp.float32),
        in_specs=[
            pl.BlockSpec(memory_space=pltpu.VMEM),
            pl.BlockSpec(memory_space=pltpu.VMEM),
            pl.BlockSpec(memory_space=pltpu.ANY),
            pl.BlockSpec(memory_space=pltpu.ANY),
            pl.BlockSpec(memory_space=pltpu.VMEM),
        ],
        out_specs=pl.BlockSpec(memory_space=pltpu.VMEM),
        scratch_shapes=[
            pltpu.VMEM((2, 2, DM, DM), jnp.float32),
            pltpu.VMEM((2, HQ, SKV, DH), jnp.float32),
            pltpu.VMEM((2, HQ, SKV, DH), jnp.float32),
            pltpu.SemaphoreType.DMA((2,)),
            pltpu.SemaphoreType.DMA((2,)),
            pltpu.SemaphoreType.DMA((2,)),
            pltpu.SemaphoreType.REGULAR,
        ],
        compiler_params=pltpu.CompilerParams(collective_id=0),
    )(x2, Wq, K, V, Wo)
    return out[None]


# baseline (device time: 806504 ns/iter reference)
import jax
import jax.numpy as jnp
from jax import lax
from jax.experimental import pallas as pl
from jax.experimental.pallas import tpu as pltpu

N_DEV = 8
SQ = 1024
SKV = 1024
HQ = 8
DH = 128
DM = 1024
SCALE = 0.08838834764831843


def kernel(x, Wq, K_ext, V_ext, Wo):
    idx = lax.axis_index("i")
    x2 = x[0]
    K = lax.dynamic_index_in_dim(K_ext, idx, axis=0, keepdims=False)
    V = lax.dynamic_index_in_dim(V_ext, idx, axis=0, keepdims=False)
    K = K.transpose(1, 0, 2)
    V = V.transpose(1, 0, 2)

    def body(x_ref, wq_ref, k_ref, v_ref, wo_ref, out_ref,
             comm_ref, k_buf, v_buf, q_buf, send_sems, recv_sems, local_sems,
             credit_sem):
        my = lax.axis_index("i")
        left = lax.rem(my + (N_DEV - 1), N_DEV)
        right = lax.rem(my + 1, N_DEV)

        barrier_sem = pltpu.get_barrier_semaphore()
        for nbr in (left, right):
            pl.semaphore_signal(barrier_sem, inc=1, device_id=(nbr,),
                                device_id_type=pl.DeviceIdType.MESH)
        pl.semaphore_wait(barrier_sem, 2)

        qi = lax.broadcasted_iota(jnp.int32, (SQ, SKV), 0)
        ki = lax.broadcasted_iota(jnp.int32, (SQ, SKV), 1)
        mask = (jnp.abs(qi - ki) <= 128) | (ki < 32) | (qi < 32)
        bias = jnp.where(mask, 0.0, -1e9).astype(jnp.float32)

        comm_ref[0, 0] = wq_ref[...]
        comm_ref[0, 1] = wo_ref[...]
        out_ref[...] = jnp.zeros((SQ, DM), jnp.float32)

        for h in range(N_DEV):
            slot = h % 2
            hb = lax.rem(my + (N_DEV - h), N_DEV)

            if h > 0:
                in_rdma = pltpu.make_async_remote_copy(
                    src_ref=comm_ref.at[slot],
                    dst_ref=comm_ref.at[slot],
                    send_sem=send_sems.at[slot],
                    recv_sem=recv_sems.at[slot],
                    device_id=(left,),
                    device_id_type=pl.DeviceIdType.MESH,
                )
                in_rdma.wait_recv()

            if h < N_DEV - 1:
                if h >= 1:
                    pl.semaphore_wait(credit_sem, 1)
                out_rdma = pltpu.make_async_remote_copy(
                    src_ref=comm_ref.at[slot],
                    dst_ref=comm_ref.at[(h + 1) % 2],
                    send_sem=send_sems.at[slot],
                    recv_sem=recv_sems.at[(h + 1) % 2],
                    device_id=(right,),
                    device_id_type=pl.DeviceIdType.MESH,
                )
                out_rdma.start()

            kcp = pltpu.make_async_copy(
                k_ref.at[pl.ds(hb * HQ, HQ)], k_buf.at[slot], local_sems.at[0])
            vcp = pltpu.make_async_copy(
                v_ref.at[pl.ds(hb * HQ, HQ)], v_buf.at[slot], local_sems.at[1])
            kcp.start()
            vcp.start()
            kcp.wait()
            vcp.wait()

            q_buf[...] = jnp.dot(x_ref[...], comm_ref[slot, 0],
                                 preferred_element_type=jnp.float32)

            def head_body(head, _):
                qh = q_buf[:, pl.ds(head * DH, DH)]
                kh = k_buf[slot, head]
                s = lax.dot_general(
                    qh, kh, (((1,), (1,)), ((), ())),
                    preferred_element_type=jnp.float32) * SCALE + bias
                m = jnp.max(s, axis=1, keepdims=True)
                w = jnp.exp(s - m)
                w = w / jnp.sum(w, axis=1, keepdims=True)
                ctx_h = jnp.dot(w, v_buf[slot, head],
                                preferred_element_type=jnp.float32)
                wo_rows = comm_ref[slot, 1, pl.ds(head * DH, DH)]
                out_ref[...] = out_ref[...] + jnp.dot(
                    ctx_h, wo_rows, preferred_element_type=jnp.float32)
                return 0

            lax.fori_loop(0, HQ, head_body, 0)

            if h < N_DEV - 1:
                out_rdma.wait_send()
            if h <= N_DEV - 3:
                pl.semaphore_signal(credit_sem, inc=1, device_id=(left,),
                                    device_id_type=pl.DeviceIdType.MESH)

    out = pl.pallas_call(
        body,
        out_shape=jax.ShapeDtypeStruct((SQ, DM), jnp.float32),
        in_specs=[
            pl.BlockSpec(memory_space=pltpu.VMEM),
            pl.BlockSpec(memory_space=pltpu.VMEM),
            pl.BlockSpec(memory_space=pl.ANY),
            pl.BlockSpec(memory_space=pl.ANY),
            pl.BlockSpec(memory_space=pltpu.VMEM),
        ],
        out_specs=pl.BlockSpec(memory_space=pltpu.VMEM),
        scratch_shapes=[
            pltpu.VMEM((2, 2, DM, DM), jnp.float32),
            pltpu.VMEM((2, HQ, SKV, DH), jnp.float32),
            pltpu.VMEM((2, HQ, SKV, DH), jnp.float32),
            pltpu.VMEM((SQ, DM), jnp.float32),
            pltpu.SemaphoreType.DMA((2,)),
            pltpu.SemaphoreType.DMA((2,)),
            pltpu.SemaphoreType.DMA((2,)),
            pltpu.SemaphoreType.REGULAR,
        ],
        compiler_params=pltpu.CompilerParams(
            collective_id=0, vmem_limit_bytes=64 * 1024 * 1024),
    )(x2, Wq, K, V, Wo)
    return out[None]


# device time: 458971 ns/iter; 1.7572x vs baseline; 1.7572x over previous
import jax
import jax.numpy as jnp
from jax import lax
from jax.experimental import pallas as pl
from jax.experimental.pallas import tpu as pltpu

N_DEV = 8
SQ = 1024
SKV = 1024
HQ = 8
DH = 128
DM = 1024
SCALE = 0.08838834764831843


def kernel(x, Wq, K_ext, V_ext, Wo):
    idx = lax.axis_index("i")
    x2 = x[0].astype(jnp.bfloat16)
    K = lax.dynamic_index_in_dim(K_ext, idx, axis=0, keepdims=False)
    V = lax.dynamic_index_in_dim(V_ext, idx, axis=0, keepdims=False)
    K = K.transpose(1, 0, 2).astype(jnp.bfloat16)
    V = V.transpose(1, 0, 2).astype(jnp.bfloat16)
    Wq = Wq.astype(jnp.bfloat16)
    Wo = Wo.astype(jnp.bfloat16)

    def body(x_ref, wq_ref, k_ref, v_ref, wo_ref, out_ref,
             comm_ref, k_buf, v_buf, q_buf, ctx_buf,
             send_sems, recv_sems, local_sems, credit_sem):
        my = lax.axis_index("i")
        left = lax.rem(my + (N_DEV - 1), N_DEV)
        right = lax.rem(my + 1, N_DEV)

        barrier_sem = pltpu.get_barrier_semaphore()
        for nbr in (left, right):
            pl.semaphore_signal(barrier_sem, inc=1, device_id=(nbr,),
                                device_id_type=pl.DeviceIdType.MESH)
        pl.semaphore_wait(barrier_sem, 2)

        qi = lax.broadcasted_iota(jnp.int32, (SQ, SKV), 0)
        ki = lax.broadcasted_iota(jnp.int32, (SQ, SKV), 1)
        mask = (jnp.abs(qi - ki) <= 128) | (ki < 32) | (qi < 32)
        bias = jnp.where(mask, 0.0, -1e9).astype(jnp.float32)

        comm_ref[0, 0] = wq_ref[...]
        comm_ref[0, 1] = wo_ref[...]
        out_ref[...] = jnp.zeros((SQ, DM), jnp.float32)

        for h in range(N_DEV):
            slot = h % 2
            hb = lax.rem(my + (N_DEV - h), N_DEV)

            if h > 0:
                in_rdma = pltpu.make_async_remote_copy(
                    src_ref=comm_ref.at[slot],
                    dst_ref=comm_ref.at[slot],
                    send_sem=send_sems.at[slot],
                    recv_sem=recv_sems.at[slot],
                    device_id=(left,),
                    device_id_type=pl.DeviceIdType.MESH,
                )
                in_rdma.wait_recv()

            if h < N_DEV - 1:
                if h >= 1:
                    pl.semaphore_wait(credit_sem, 1)
                out_rdma = pltpu.make_async_remote_copy(
                    src_ref=comm_ref.at[slot],
                    dst_ref=comm_ref.at[(h + 1) % 2],
                    send_sem=send_sems.at[slot],
                    recv_sem=recv_sems.at[(h + 1) % 2],
                    device_id=(right,),
                    device_id_type=pl.DeviceIdType.MESH,
                )
                out_rdma.start()

            kcp = pltpu.make_async_copy(
                k_ref.at[pl.ds(hb * HQ, HQ)], k_buf.at[slot], local_sems.at[0])
            vcp = pltpu.make_async_copy(
                v_ref.at[pl.ds(hb * HQ, HQ)], v_buf.at[slot], local_sems.at[1])
            kcp.start()
            vcp.start()
            kcp.wait()
            vcp.wait()

            q_buf[...] = jnp.dot(
                x_ref[...], comm_ref[slot, 0],
                preferred_element_type=jnp.float32).astype(jnp.bfloat16)

            def head_body(head, _):
                qh = q_buf[:, pl.ds(head * DH, DH)]
                kh = k_buf[slot, head]
                s = lax.dot_general(
                    qh, kh, (((1,), (1,)), ((), ())),
                    preferred_element_type=jnp.float32) * SCALE + bias
                m = jnp.max(s, axis=1, keepdims=True)
                w = jnp.exp(s - m)
                w = (w / jnp.sum(w, axis=1, keepdims=True)).astype(jnp.bfloat16)
                ctx_buf[:, pl.ds(head * DH, DH)] = jnp.dot(
                    w, v_buf[slot, head],
                    preferred_element_type=jnp.float32).astype(jnp.bfloat16)
                return 0

            lax.fori_loop(0, HQ, head_body, 0)

            out_ref[...] = out_ref[...] + jnp.dot(
                ctx_buf[...], comm_ref[slot, 1],
                preferred_element_type=jnp.float32)

            if h < N_DEV - 1:
                out_rdma.wait_send()
            if h <= N_DEV - 3:
                pl.semaphore_signal(credit_sem, inc=1, device_id=(left,),
                                    device_id_type=pl.DeviceIdType.MESH)

    out = pl.pallas_call(
        body,
        out_shape=jax.ShapeDtypeStruct((SQ, DM), jnp.float32),
        in_specs=[
            pl.BlockSpec(memory_space=pltpu.VMEM),
            pl.BlockSpec(memory_space=pltpu.VMEM),
            pl.BlockSpec(memory_space=pl.ANY),
            pl.BlockSpec(memory_space=pl.ANY),
            pl.BlockSpec(memory_space=pltpu.VMEM),
        ],
        out_specs=pl.BlockSpec(memory_space=pltpu.VMEM),
        scratch_shapes=[
            pltpu.VMEM((2, 2, DM, DM), jnp.bfloat16),
            pltpu.VMEM((2, HQ, SKV, DH), jnp.bfloat16),
            pltpu.VMEM((2, HQ, SKV, DH), jnp.bfloat16),
            pltpu.VMEM((SQ, DM), jnp.bfloat16),
            pltpu.VMEM((SQ, DM), jnp.bfloat16),
            pltpu.SemaphoreType.DMA((2,)),
            pltpu.SemaphoreType.DMA((2,)),
            pltpu.SemaphoreType.DMA((2,)),
            pltpu.SemaphoreType.REGULAR,
        ],
        compiler_params=pltpu.CompilerParams(
            collective_id=0, vmem_limit_bytes=64 * 1024 * 1024),
    )(x2, Wq, K, V, Wo)
    return out[None]
